# baseline (speedup 1.0000x reference)
import jax
import jax.numpy as jnp
from jax import lax
from jax.experimental import pallas as pl
from jax.experimental.pallas import tpu as pltpu
from jax.experimental.pallas import tpu_sc as plsc

_L = 16
_B, _P, _H, _W = 16, 3000, 256, 256
_PP = 3072
_HALF = _PP // 2
_STEPS = _HALF // _L


_UNROLL = 4


def _bf16_gather(map_ref, idx):
    word = plsc.load_gather(map_ref, [lax.shift_right_logical(idx, 1)])
    odd = (idx & 1) == 1
    bits = jnp.where(odd, word & jnp.int32(-65536), word << 16)
    return plsc.bitcast(bits, jnp.float32)


def _chunk(map_ref, w_ref, t_ref, off):
    w = w_ref[pl.ds(off, _L)]
    idx_a = (w & 255) * _W + ((w >> 8) & 255)
    idx_b = ((w >> 16) & 255) * _W + ((w >> 24) & 255)
    za = _bf16_gather(map_ref, idx_a)
    zb = _bf16_gather(map_ref, idx_b)
    t = t_ref[pl.ds(off, _L)]
    u = t * (za - zb)
    e = jnp.exp(-jnp.abs(u))
    z = e / (2.0 + e)
    z2 = z * z
    p = 2.0 * z * (1.0 + z2 * (1.0 / 3.0 + z2 * (0.2 + z2 * (1.0 / 7.0 + z2 * (1.0 / 9.0)))))
    val = jnp.maximum(u, 0.0) + p
    m = t != 0.0
    return jnp.where(m, val, 0.0), jnp.where(m, 1.0, 0.0)


def _softplus_steps(map_ref, w_ref, t_ref):
    def body(j, carry):
        accs = list(carry)
        base = j * (_L * _UNROLL)
        for k in range(_UNROLL):
            v, c = _chunk(map_ref, w_ref, t_ref, base + k * _L)
            accs[2 * k] = accs[2 * k] + v
            accs[2 * k + 1] = accs[2 * k + 1] + c
        return tuple(accs)

    zero = jnp.zeros((_L,), jnp.float32)
    accs = lax.fori_loop(0, _STEPS // _UNROLL, body, (zero,) * (2 * _UNROLL))
    s_vec = accs[0]
    c_vec = accs[1]
    for k in range(1, _UNROLL):
        s_vec = s_vec + accs[2 * k]
        c_vec = c_vec + accs[2 * k + 1]
    return s_vec, c_vec


def _sc_body(flat_hbm, w_hbm, t_hbm, sums_hbm, cnts_hbm,
             map_v, w_v, t_v, res_s, res_c, sem1, sem2, sem3):
    batch = lax.axis_index("s")
    half = lax.axis_index("c")
    base = half * _HALF
    h1 = pltpu.async_copy(flat_hbm.at[batch], map_v, sem1)
    h2 = pltpu.async_copy(w_hbm.at[batch, pl.ds(base, _HALF)], w_v, sem2)
    h3 = pltpu.async_copy(t_hbm.at[batch, pl.ds(base, _HALF)], t_v, sem3)
    h2.wait()
    h3.wait()
    h1.wait()
    s_vec, c_vec = _softplus_steps(map_v, w_v, t_v)
    res_s[...] = s_vec
    res_c[...] = c_vec
    pltpu.sync_copy(res_s, sums_hbm.at[batch, pl.ds(half * _L, _L)])
    pltpu.sync_copy(res_c, cnts_hbm.at[batch, pl.ds(half * _L, _L)])


@jax.jit
def _sc_partials(flat, w, t):
    mesh = plsc.VectorSubcoreMesh(core_axis_name="c", subcore_axis_name="s")
    return pl.kernel(
        _sc_body,
        out_type=[
            jax.ShapeDtypeStruct((_B, 2 * _L), jnp.float32),
            jax.ShapeDtypeStruct((_B, 2 * _L), jnp.float32),
        ],
        mesh=mesh,
        compiler_params=pltpu.CompilerParams(needs_layout_passes=False),
        scratch_types=[
            pltpu.VMEM((_H * _W // 2,), jnp.int32),
            pltpu.VMEM((_HALF,), jnp.int32),
            pltpu.VMEM((_HALF,), jnp.float32),
            pltpu.VMEM((_L,), jnp.float32),
            pltpu.VMEM((_L,), jnp.float32),
            pltpu.SemaphoreType.DMA,
            pltpu.SemaphoreType.DMA,
            pltpu.SemaphoreType.DMA,
        ],
    )(flat, w, t)


def _combine_body(s_ref, c_ref, o_ref):
    s = jnp.sum(s_ref[...], axis=1)
    c = jnp.sum(c_ref[...], axis=1)
    per = s / jnp.maximum(c, 1.0)
    o_ref[...] = (jnp.sum(per) / _B).reshape(1, 1)


@jax.jit
def _combine(sums, cnts):
    return pl.pallas_call(
        _combine_body,
        out_shape=jax.ShapeDtypeStruct((1, 1), jnp.float32),
    )(sums, cnts)


def kernel(output, x_A, y_A, x_B, y_B, ordinal_relation):
    fb = output.reshape(_B, _H * _W).astype(jnp.bfloat16)
    flat = jax.lax.bitcast_convert_type(fb.reshape(_B, _H * _W // 2, 2), jnp.int32)
    w = ((x_A & 255)
         | ((y_A & 255) << 8)
         | ((x_B & 255) << 16)
         | ((y_B & 255) << 24)).astype(jnp.int32)
    pad = ((0, 0), (0, _PP - _P))
    w = jnp.pad(w, pad)
    t = jnp.pad(ordinal_relation, pad)
    sums, cnts = _sc_partials(flat, w, t)
    return _combine(sums, cnts)[0, 0]


# speedup vs baseline: 7.4779x; 7.4779x over previous
import jax
import jax.numpy as jnp
from jax import lax
from jax.experimental import pallas as pl
from jax.experimental.pallas import tpu as pltpu
from jax.experimental.pallas import tpu_sc as plsc

_L = 16
_B, _P, _H, _W = 16, 3000, 256, 256
_PP = 3072
_HALF = _PP // 2
_STEPS = _HALF // _L


_UNROLL = 4


def _chunk(map_ref, w_ref, t_ref, off):
    w = w_ref[pl.ds(off, _L)]
    idx_a = (w & 255) * _W + ((w >> 8) & 255)
    idx_b = ((w >> 16) & 255) * _W + ((w >> 24) & 255)
    za = plsc.load_gather(map_ref, [idx_a])
    zb = plsc.load_gather(map_ref, [idx_b])
    t = t_ref[pl.ds(off, _L)]
    u = t * (za - zb)
    e = jnp.exp(-jnp.abs(u))
    z = e / (2.0 + e)
    z2 = z * z
    p = 2.0 * z * (1.0 + z2 * (1.0 / 3.0 + z2 * (0.2 + z2 * (1.0 / 7.0 + z2 * (1.0 / 9.0)))))
    val = jnp.maximum(u, 0.0) + p
    m = t != 0.0
    return jnp.where(m, val, 0.0), jnp.where(m, 1.0, 0.0)


def _softplus_steps(map_ref, w_ref, t_ref):
    def body(j, carry):
        accs = list(carry)
        base = j * (_L * _UNROLL)
        for k in range(_UNROLL):
            v, c = _chunk(map_ref, w_ref, t_ref, base + k * _L)
            accs[2 * k] = accs[2 * k] + v
            accs[2 * k + 1] = accs[2 * k + 1] + c
        return tuple(accs)

    zero = jnp.zeros((_L,), jnp.float32)
    accs = lax.fori_loop(0, _STEPS // _UNROLL, body, (zero,) * (2 * _UNROLL))
    s_vec = accs[0]
    c_vec = accs[1]
    for k in range(1, _UNROLL):
        s_vec = s_vec + accs[2 * k]
        c_vec = c_vec + accs[2 * k + 1]
    return s_vec, c_vec


def _sc_body(flat_hbm, w_hbm, t_hbm, sums_hbm, cnts_hbm,
             map_v, w_v, t_v, res_s, res_c, sem1, sem2, sem3):
    batch = lax.axis_index("s")
    half = lax.axis_index("c")
    base = half * _HALF
    h1 = pltpu.async_copy(flat_hbm.at[batch], map_v, sem1)
    h2 = pltpu.async_copy(w_hbm.at[batch, pl.ds(base, _HALF)], w_v, sem2)
    h3 = pltpu.async_copy(t_hbm.at[batch, pl.ds(base, _HALF)], t_v, sem3)
    h2.wait()
    h3.wait()
    h1.wait()
    s_vec, c_vec = _softplus_steps(map_v, w_v, t_v)
    res_s[...] = s_vec
    res_c[...] = c_vec
    pltpu.sync_copy(res_s, sums_hbm.at[batch, pl.ds(half * _L, _L)])
    pltpu.sync_copy(res_c, cnts_hbm.at[batch, pl.ds(half * _L, _L)])


@jax.jit
def _sc_partials(flat, w, t):
    mesh = plsc.VectorSubcoreMesh(core_axis_name="c", subcore_axis_name="s")
    return pl.kernel(
        _sc_body,
        out_type=[
            jax.ShapeDtypeStruct((_B, 2 * _L), jnp.float32),
            jax.ShapeDtypeStruct((_B, 2 * _L), jnp.float32),
        ],
        mesh=mesh,
        compiler_params=pltpu.CompilerParams(needs_layout_passes=False),
        scratch_types=[
            pltpu.VMEM((_H * _W,), jnp.float32),
            pltpu.VMEM((_HALF,), jnp.int32),
            pltpu.VMEM((_HALF,), jnp.float32),
            pltpu.VMEM((_L,), jnp.float32),
            pltpu.VMEM((_L,), jnp.float32),
            pltpu.SemaphoreType.DMA,
            pltpu.SemaphoreType.DMA,
            pltpu.SemaphoreType.DMA,
        ],
    )(flat, w, t)


def _combine_body(s_ref, c_ref, o_ref):
    s = jnp.sum(s_ref[...], axis=1)
    c = jnp.sum(c_ref[...], axis=1)
    per = s / jnp.maximum(c, 1.0)
    o_ref[...] = (jnp.sum(per) / _B).reshape(1, 1)


@jax.jit
def _combine(sums, cnts):
    return pl.pallas_call(
        _combine_body,
        out_shape=jax.ShapeDtypeStruct((1, 1), jnp.float32),
    )(sums, cnts)


def kernel(output, x_A, y_A, x_B, y_B, ordinal_relation):
    flat = output.reshape(_B, _H * _W)
    w = ((x_A & 255)
         | ((y_A & 255) << 8)
         | ((x_B & 255) << 16)
         | ((y_B & 255) << 24)).astype(jnp.int32)
    pad = ((0, 0), (0, _PP - _P))
    w = jnp.pad(w, pad)
    t = jnp.pad(ordinal_relation, pad)
    sums, cnts = _sc_partials(flat, w, t)
    return _combine(sums, cnts)[0, 0]
